# baseline (speedup 1.0000x reference)
import functools

import jax
import jax.numpy as jnp
from jax import lax
from jax.experimental import pallas as pl
from jax.experimental.pallas import tpu as pltpu
from jax.experimental.pallas import tpu_sc as plsc

N_NODES = 65536
N_EDGES = 1048576
N_POLY = 4096
BATCH = 32
TSL = 128
HID = 64

NC = 2
NS = 16
NW = NC * NS
NBKT = 64
BKT_SHIFT = 10
BKT_SZ = N_NODES // NBKT
EPT = N_EDGES // NW
PAD = 2048
ECH = 1024
GSUB = 128
ROWS_PER_TC_BLK = 2048

_mesh = plsc.VectorSubcoreMesh(core_axis_name="c", subcore_axis_name="s")


def _wid():
    return lax.axis_index("s") * NC + lax.axis_index("c")


RANK_BASE = 1

TSTRIDE = 34816
def _align16(x):
    return jnp.bitwise_and(x + 15, -16)


@functools.partial(
    pl.kernel, mesh=_mesh,
    compiler_params=pltpu.CompilerParams(needs_layout_passes=False, use_tc_tiling_on_sc=False),
    out_type=(jax.ShapeDtypeStruct((NW * TSTRIDE + 2048,), jnp.int32),
              jax.ShapeDtypeStruct((NW, NBKT), jnp.int32)),
    scratch_types=[
        pltpu.VMEM((2048,), jnp.int32),
        pltpu.VMEM((2048,), jnp.int32),
        pltpu.VMEM((EPT,), jnp.int32),
        pltpu.VMEM((TSTRIDE,), jnp.int32),
        pltpu.VMEM((NBKT,), jnp.int32),
        pltpu.VMEM((NBKT,), jnp.int32),
    ],
)
def _e13_bucket(edge_hbm, bucketed_hbm, counts_hbm,
                src_v, dst_v, pk_v, stg_v, cnt_v, cur_v):
    w = _wid()
    base = w * EPT
    for i in range(NBKT // 16):
        cnt_v[pl.ds(i * 16, 16)] = jnp.zeros((16,), jnp.int32)

    def chunk_body(c, _):
        off = pl.multiple_of(base + c * 2048, 2048)
        pltpu.sync_copy(edge_hbm.at[0, pl.ds(off, 2048)], src_v)
        pltpu.sync_copy(edge_hbm.at[1, pl.ds(off, 2048)], dst_v)

        def grp(i, _):
            s = src_v[pl.ds(i * 16, 16)]
            d = dst_v[pl.ds(i * 16, 16)]
            pk_v[pl.ds(c * 2048 + i * 16, 16)] = jnp.bitwise_or(
                s, jnp.left_shift(d, 16))
            b = lax.shift_right_logical(d, BKT_SHIFT)
            rank, last = plsc.scan_count(b)
            bse = plsc.load_gather(cnt_v, [b])
            plsc.store_scatter(cnt_v, [b], bse + rank + (1 - RANK_BASE),
                               mask=last)
            return 0

        lax.fori_loop(0, 2048 // 16, grp, 0)
        return 0

    lax.fori_loop(0, EPT // 2048, chunk_body, 0)

    def initgrp(g, running):
        sl = pl.ds(g * 16, 16)
        tot16 = cnt_v[sl]
        ptot16 = _align16(tot16)
        pcum = plsc.cumsum(ptot16)
        cur_v[sl] = running + (pcum - ptot16)
        return running + pcum[15]

    lax.fori_loop(0, NBKT // 16, initgrp, 0)

    def rank(i, _):
        pk = pk_v[pl.ds(i * 16, 16)]
        b = lax.shift_right_logical(pk, 16 + BKT_SHIFT)
        rank16, last = plsc.scan_count(b)
        bse = plsc.load_gather(cur_v, [b])
        plsc.store_scatter(cur_v, [b], bse + rank16 + (1 - RANK_BASE),
                           mask=last)
        plsc.store_scatter(stg_v, [bse + rank16 - RANK_BASE], pk)
        return 0

    lax.fori_loop(0, EPT // 16, rank, 0)

    pltpu.sync_copy(
        stg_v, bucketed_hbm.at[pl.ds(pl.multiple_of(w * TSTRIDE, 16),
                                     TSTRIDE)])
    pltpu.sync_copy(cnt_v, counts_hbm.at[w])


def _seg_off(cnts_v, wp, b):
    lane = jnp.bitwise_and(b, 15)
    grp = lax.shift_right_logical(b, 4)
    lanes = lax.iota(jnp.int32, 16)

    def gbody(g, carry):
        running, start, plen, rcnt = carry
        row16 = cnts_v[wp, pl.ds(g * 16, 16)]
        p16 = _align16(row16)
        pcum = plsc.cumsum(p16)
        here = jnp.logical_and(g == grp, lanes == lane)
        start = start + jnp.sum(jnp.where(here, pcum - p16 + running, 0))
        plen = plen + jnp.sum(jnp.where(here, p16, 0))
        rcnt = rcnt + jnp.sum(jnp.where(here, row16, 0))
        return (running + pcum[15], start, plen, rcnt)

    _, start, plen, rcnt = lax.fori_loop(0, NBKT // 16, gbody, (0, 0, 0, 0))
    return start, plen, rcnt


@functools.partial(
    pl.kernel, mesh=_mesh,
    compiler_params=pltpu.CompilerParams(needs_layout_passes=False, use_tc_tiling_on_sc=False),
    out_type=jax.ShapeDtypeStruct((N_NODES, HID), jnp.float32),
    scratch_types=[
        pltpu.VMEM((BKT_SZ + 1, HID), jnp.float32),
        pltpu.VMEM((ECH + 32,), jnp.int32),
        pltpu.VMEM((ECH // GSUB, GSUB), jnp.int32),
        pltpu.VMEM((4, GSUB, HID), jnp.float32),
        pltpu.VMEM((NW, NBKT), jnp.int32),
        pltpu.SemaphoreType.DMA,
    ],
)
def _e4_segmax(h_hbm, bucketed_hbm, counts_hbm, agg_hbm,
               agg_v, ebuf_v, gidx_v, rows_v, cnts_v, sem):
    w = _wid()
    pltpu.sync_copy(counts_hbm, cnts_v)

    def bucket_body(rb, _):
        b = w + NW * rb

        def zrow(i, _):
            for q in range(HID // 16):
                agg_v[i, pl.ds(q * 16, 16)] = jnp.zeros((16,), jnp.float32)
            return 0

        lax.fori_loop(0, BKT_SZ + 1, zrow, 0)

        def seg_body(wp, _):
            soff, plen, rcnt = _seg_off(cnts_v, wp, b)
            seg_base = wp * TSTRIDE + soff
            nch = lax.div(plen + (ECH - 1), ECH)

            def chunk_body(c, _):
                cbase = pl.multiple_of(seg_base + c * ECH, 8)
                pltpu.sync_copy(bucketed_hbm.at[pl.ds(cbase, ECH + 16)],
                                ebuf_v.at[pl.ds(0, ECH + 16)])
                rem = jnp.minimum(ECH, plen - c * ECH)
                erem = rcnt - c * ECH
                nsub = lax.div(rem + (GSUB - 1), GSUB)

                def build_idx(j):
                    def bi(i, _):
                        p16 = ebuf_v[pl.ds(j * GSUB + i * 16, 16)]
                        gidx_v[j, pl.ds(i * 16, 16)] = jnp.bitwise_and(
                            p16, 0xFFFF)
                        return 0
                    lax.fori_loop(0, GSUB // 16, bi, 0)

                def consume(j):
                    ngrp = lax.div(jnp.minimum(GSUB, rem - j * GSUB), 16)
                    jb = jnp.bitwise_and(j, 3)

                    lanes16 = lax.iota(jnp.int32, 16)

                    def cg(i, _):
                        p16 = ebuf_v[pl.ds(j * GSUB + i * 16, 16)]
                        d16 = jnp.bitwise_and(
                            lax.shift_right_logical(p16, 16), BKT_SZ - 1)
                        valid16 = (j * GSUB + i * 16 + lanes16) < erem
                        d16 = jnp.where(valid16, d16, BKT_SZ)
                        for l in range(16):
                            d = d16[l]
                            for q in range(HID // 16):
                                sl = pl.ds(q * 16, 16)
                                agg_v[d, sl] = jnp.maximum(
                                    agg_v[d, sl],
                                    rows_v[jb, i * 16 + l, sl])
                        return 0

                    lax.fori_loop(0, ngrp, cg, 0)

                build_idx(0)
                pltpu.async_copy(h_hbm.at[gidx_v.at[0]], rows_v.at[0], sem)
                for pj in (1, 2):
                    @pl.when(pj < nsub)
                    def _prefill(pj=pj):
                        build_idx(pj)
                        pltpu.async_copy(
                            h_hbm.at[gidx_v.at[pj]], rows_v.at[pj], sem)

                def sub_body(j, _):
                    @pl.when(j + 3 < nsub)
                    def _issue_next():
                        jn = j + 3
                        build_idx(jn)
                        pltpu.async_copy(
                            h_hbm.at[gidx_v.at[jn]],
                            rows_v.at[jnp.bitwise_and(jn, 3)], sem)

                    pltpu.make_async_copy(
                        h_hbm.at[pl.ds(0, GSUB)],
                        rows_v.at[jnp.bitwise_and(j, 3)], sem).wait()
                    consume(j)
                    return 0

                lax.fori_loop(0, nsub, sub_body, 0)
                return 0

            lax.fori_loop(0, nch, chunk_body, 0)
            return 0

        lax.fori_loop(0, NW, seg_body, 0)
        pltpu.sync_copy(agg_v.at[pl.ds(0, BKT_SZ)],
                        agg_hbm.at[pl.ds(b * BKT_SZ, BKT_SZ)])
        return 0

    lax.fori_loop(0, NBKT // NW, bucket_body, 0)


POLY_PER = N_POLY // NW
NEG_INIT = -1e30


@functools.partial(
    pl.kernel, mesh=_mesh,
    compiler_params=pltpu.CompilerParams(needs_layout_passes=False, use_tc_tiling_on_sc=False),
    out_type=jax.ShapeDtypeStruct((N_POLY, HID), jnp.float32),
    scratch_types=[
        pltpu.VMEM((N_NODES + 16,), jnp.int32),
        pltpu.VMEM((POLY_PER + 1, HID), jnp.float32),
        pltpu.VMEM((GSUB, HID), jnp.float32),
    ],
)
def _e5_poly(hf_hbm, cluster_hbm, poly_hbm, cl_v, agg_v, rows_v):
    w = _wid()
    pltpu.sync_copy(cluster_hbm, cl_v.at[pl.ds(0, N_NODES)])
    p0 = w * POLY_PER
    p1 = p0 + POLY_PER

    def lower_bound(target):
        def cond(st):
            return st[0] < st[1]

        lanes = lax.iota(jnp.int32, 16)

        def body(st):
            l, r = st
            m = lax.div(l + r, 2)
            m_al = pl.multiple_of(jnp.bitwise_and(m, -16), 16)
            v16 = cl_v[pl.ds(m_al, 16)]
            cm = jnp.sum(jnp.where(lanes == (m - m_al), v16, 0))
            go_right = cm < target
            return (jnp.where(go_right, m + 1, l), jnp.where(go_right, r, m))

        return lax.while_loop(cond, body, (0, N_NODES))[0]

    lo = lower_bound(p0)
    hi = lower_bound(p1)

    def zrow(i, _):
        for q in range(HID // 16):
            agg_v[i, pl.ds(q * 16, 16)] = jnp.full((16,), NEG_INIT, jnp.float32)
        return 0

    lax.fori_loop(0, POLY_PER + 1, zrow, 0)

    lo_al = pl.multiple_of(jnp.bitwise_and(lo, -16), 16)
    nch = lax.div(hi - lo_al + (GSUB - 1), GSUB)
    lanes = lax.iota(jnp.int32, 16)

    def chunk_body(c, _):
        cbase = pl.multiple_of(lo_al + c * GSUB, 16)
        pltpu.sync_copy(hf_hbm.at[pl.ds(cbase, GSUB)], rows_v)

        def cg(i, _):
            c16 = cl_v[pl.ds(cbase + i * 16, 16)]
            for l in range(16):
                n = cbase + i * 16 + l
                valid = jnp.logical_and(n >= lo, n < hi)
                d = jnp.where(valid, c16[l] - p0, POLY_PER)
                for q in range(HID // 16):
                    sl = pl.ds(q * 16, 16)
                    agg_v[d, sl] = jnp.maximum(agg_v[d, sl],
                                               rows_v[i * 16 + l, sl])
            return 0

        lax.fori_loop(0, GSUB // 16, cg, 0)
        return 0

    lax.fori_loop(0, nch, chunk_body, 0)
    pltpu.sync_copy(agg_v.at[pl.ds(0, POLY_PER)],
                    poly_hbm.at[pl.ds(p0, POLY_PER)])


def _ln_relu(h, g, beta):
    mu = jnp.mean(h, axis=-1, keepdims=True)
    var = jnp.var(h, axis=-1, keepdims=True)
    h = (h - mu) / jnp.sqrt(var + 1e-5) * g + beta
    return jax.nn.relu(h)


def _mlp0_body(x_ref, w_ref, b_ref, g_ref, bt_ref, o_ref):
    h = jnp.dot(x_ref[...], w_ref[...], preferred_element_type=jnp.float32)
    o_ref[...] = _ln_relu(h + b_ref[...], g_ref[...], bt_ref[...])


def _mlp0(x, W0, b0, g0, beta0):
    grid = (N_NODES // ROWS_PER_TC_BLK,)
    return pl.pallas_call(
        _mlp0_body,
        grid=grid,
        in_specs=[
            pl.BlockSpec((ROWS_PER_TC_BLK, 8), lambda i: (i, 0)),
            pl.BlockSpec((8, HID), lambda i: (0, 0)),
            pl.BlockSpec((1, HID), lambda i: (0, 0)),
            pl.BlockSpec((1, HID), lambda i: (0, 0)),
            pl.BlockSpec((1, HID), lambda i: (0, 0)),
        ],
        out_specs=pl.BlockSpec((ROWS_PER_TC_BLK, HID), lambda i: (i, 0)),
        out_shape=jax.ShapeDtypeStruct((N_NODES, HID), jnp.float32),
    )(x, W0, b0.reshape(1, HID), g0.reshape(1, HID), beta0.reshape(1, HID))


def _mlp12_body(h_ref, a_ref, wa_ref, wb_ref, b_ref, g_ref, bt_ref, o_ref):
    h = (jnp.dot(h_ref[...], wa_ref[...], preferred_element_type=jnp.float32)
         + jnp.dot(a_ref[...], wb_ref[...], preferred_element_type=jnp.float32))
    o_ref[...] = _ln_relu(h + b_ref[...], g_ref[...], bt_ref[...])


def _mlp12(h, agg, W, b, g, beta):
    grid = (N_NODES // ROWS_PER_TC_BLK,)
    return pl.pallas_call(
        _mlp12_body,
        grid=grid,
        in_specs=[
            pl.BlockSpec((ROWS_PER_TC_BLK, HID), lambda i: (i, 0)),
            pl.BlockSpec((ROWS_PER_TC_BLK, HID), lambda i: (i, 0)),
            pl.BlockSpec((HID, HID), lambda i: (0, 0)),
            pl.BlockSpec((HID, HID), lambda i: (0, 0)),
            pl.BlockSpec((1, HID), lambda i: (0, 0)),
            pl.BlockSpec((1, HID), lambda i: (0, 0)),
            pl.BlockSpec((1, HID), lambda i: (0, 0)),
        ],
        out_specs=pl.BlockSpec((ROWS_PER_TC_BLK, HID), lambda i: (i, 0)),
        out_shape=jax.ShapeDtypeStruct((N_NODES, HID), jnp.float32),
    )(h, agg, W[:HID], W[HID:], b.reshape(1, HID), g.reshape(1, HID),
      beta.reshape(1, HID))


HF_ROWS = N_NODES + 2048


def _final_body(h_ref, a_ref, wa_ref, wb_ref, b_ref, o_ref):
    o_ref[...] = (
        jnp.dot(h_ref[...], wa_ref[...], preferred_element_type=jnp.float32)
        + jnp.dot(a_ref[...], wb_ref[...], preferred_element_type=jnp.float32)
        + b_ref[...])


def _final(h, agg, Wf, bf):
    grid = (N_NODES // ROWS_PER_TC_BLK,)
    return pl.pallas_call(
        _final_body,
        grid=grid,
        in_specs=[
            pl.BlockSpec((ROWS_PER_TC_BLK, HID), lambda i: (i, 0)),
            pl.BlockSpec((ROWS_PER_TC_BLK, HID), lambda i: (i, 0)),
            pl.BlockSpec((HID, HID), lambda i: (0, 0)),
            pl.BlockSpec((HID, HID), lambda i: (0, 0)),
            pl.BlockSpec((1, HID), lambda i: (0, 0)),
        ],
        out_specs=pl.BlockSpec((ROWS_PER_TC_BLK, HID), lambda i: (i, 0)),
        out_shape=jax.ShapeDtypeStruct((HF_ROWS, HID), jnp.float32),
    )(h, agg, Wf[:HID], Wf[HID:], bf.reshape(1, HID))


def _attn_body(poly_ref, id_ref, wq1_ref, wq2_ref, bq_ref, wk1_ref, wk2_ref,
               bk_ref, wv1_ref, wv2_ref, bv_ref, vl_ref, out_ref):
    p = poly_ref[...]
    p = jnp.where(p < -1e29, 0.0, p)
    nrm = jnp.sqrt(jnp.sum(p * p, axis=-1, keepdims=True))
    p = p / jnp.maximum(nrm, 1e-12)
    idb = id_ref[...]

    def proj(w1, w2, bb):
        return (jnp.dot(p, w1[...], preferred_element_type=jnp.float32)
                + idb[:, 0:1] * w2[0:1, :] + idb[:, 1:2] * w2[1:2, :]
                + bb[...])

    q = proj(wq1_ref, wq2_ref, bq_ref)
    k = proj(wk1_ref, wk2_ref, bk_ref)
    v = proj(wv1_ref, wv2_ref, bv_ref)
    s = jnp.dot(q, k.T, preferred_element_type=jnp.float32) / jnp.sqrt(
        jnp.float32(HID))
    key_pos = jax.lax.broadcasted_iota(jnp.int32, (TSL, TSL), 1).astype(
        jnp.float32)
    mask = key_pos < vl_ref[0, 0, 0]
    s = jnp.where(mask, s, -1e9)
    s = s - jnp.max(s, axis=-1, keepdims=True)
    e = jnp.exp(s)
    attn = e / jnp.sum(e, axis=-1, keepdims=True)
    out_ref[0] = jnp.dot(attn, v, preferred_element_type=jnp.float32)


def _attention(poly, identifier, Wq, bq, Wk, bk, Wv, bv, vl_f):
    full = lambda shape: pl.BlockSpec(shape, lambda b: tuple(0 for _ in shape))
    return pl.pallas_call(
        _attn_body,
        grid=(BATCH,),
        in_specs=[
            pl.BlockSpec((TSL, HID), lambda b: (b, 0)),
            pl.BlockSpec((TSL, 2), lambda b: (b, 0)),
            full((HID, HID)), full((2, HID)), full((1, HID)),
            full((HID, HID)), full((2, HID)), full((1, HID)),
            full((HID, HID)), full((2, HID)), full((1, HID)),
            pl.BlockSpec((1, 1, 1), lambda b: (b, 0, 0)),
        ],
        out_specs=pl.BlockSpec((1, TSL, HID), lambda b: (b, 0, 0)),
        out_shape=jax.ShapeDtypeStruct((BATCH, TSL, HID), jnp.float32),
    )(poly, identifier, Wq[:HID], Wq[HID:], bq.reshape(1, HID),
      Wk[:HID], Wk[HID:], bk.reshape(1, HID),
      Wv[:HID], Wv[HID:], bv.reshape(1, HID), vl_f)


def kernel(x, edge_index, cluster, identifier, valid_len, time_step_len,
           W0, b0, g0, beta0, W1, b1, g1, beta1, W2, b2, g2, beta2,
           Wf, bf, Wq, bq, Wk, bk, Wv, bv):
    bucketed, counts = _e13_bucket(edge_index)

    h = _mlp0(x, W0, b0, g0, beta0)
    agg = _e4_segmax(h, bucketed, counts)
    h = _mlp12(h, agg, W1, b1, g1, beta1)
    agg = _e4_segmax(h, bucketed, counts)
    h = _mlp12(h, agg, W2, b2, g2, beta2)
    agg = _e4_segmax(h, bucketed, counts)
    hf = _final(h, agg, Wf, bf)

    poly = _e5_poly(hf, cluster)

    vl_f = jnp.minimum(valid_len, time_step_len).astype(jnp.float32).reshape(
        BATCH, 1, 1)
    return _attention(poly, identifier, Wq, bq, Wk, bk, Wv, bv, vl_f)


# speedup vs baseline: 1.2927x; 1.2927x over previous
import functools

import jax
import jax.numpy as jnp
from jax import lax
from jax.experimental import pallas as pl
from jax.experimental.pallas import tpu as pltpu
from jax.experimental.pallas import tpu_sc as plsc

N_NODES = 65536
N_EDGES = 1048576
N_POLY = 4096
BATCH = 32
TSL = 128
HID = 64

NC = 2
NS = 16
NW = NC * NS
NBKT = 64
BKT_SHIFT = 10
BKT_SZ = N_NODES // NBKT
EPT = N_EDGES // NW
PAD = 2048
ECH = 1024
GSUB = 128
ROWS_PER_TC_BLK = 2048

_mesh = plsc.VectorSubcoreMesh(core_axis_name="c", subcore_axis_name="s")


def _wid():
    return lax.axis_index("s") * NC + lax.axis_index("c")


RANK_BASE = 1

TSTRIDE = 34816
def _align16(x):
    return jnp.bitwise_and(x + 15, -16)


@functools.partial(
    pl.kernel, mesh=_mesh,
    compiler_params=pltpu.CompilerParams(needs_layout_passes=False, use_tc_tiling_on_sc=False),
    out_type=(jax.ShapeDtypeStruct((NW * TSTRIDE + 2048,), jnp.int32),
              jax.ShapeDtypeStruct((NW, NBKT), jnp.int32)),
    scratch_types=[
        pltpu.VMEM((2048,), jnp.int32),
        pltpu.VMEM((2048,), jnp.int32),
        pltpu.VMEM((EPT,), jnp.int32),
        pltpu.VMEM((TSTRIDE,), jnp.int32),
        pltpu.VMEM((NBKT,), jnp.int32),
        pltpu.VMEM((NBKT,), jnp.int32),
    ],
)
def _e13_bucket(edge_hbm, bucketed_hbm, counts_hbm,
                src_v, dst_v, pk_v, stg_v, cnt_v, cur_v):
    w = _wid()
    base = w * EPT
    for i in range(NBKT // 16):
        cnt_v[pl.ds(i * 16, 16)] = jnp.zeros((16,), jnp.int32)

    def chunk_body(c, _):
        off = pl.multiple_of(base + c * 2048, 2048)
        pltpu.sync_copy(edge_hbm.at[0, pl.ds(off, 2048)], src_v)
        pltpu.sync_copy(edge_hbm.at[1, pl.ds(off, 2048)], dst_v)

        def grp(i, _):
            s = src_v[pl.ds(i * 16, 16)]
            d = dst_v[pl.ds(i * 16, 16)]
            pk_v[pl.ds(c * 2048 + i * 16, 16)] = jnp.bitwise_or(
                s, jnp.left_shift(d, 16))
            b = lax.shift_right_logical(d, BKT_SHIFT)
            rank, last = plsc.scan_count(b)
            bse = plsc.load_gather(cnt_v, [b])
            plsc.store_scatter(cnt_v, [b], bse + rank + (1 - RANK_BASE),
                               mask=last)
            return 0

        lax.fori_loop(0, 2048 // 16, grp, 0)
        return 0

    lax.fori_loop(0, EPT // 2048, chunk_body, 0)

    def initgrp(g, running):
        sl = pl.ds(g * 16, 16)
        tot16 = cnt_v[sl]
        ptot16 = _align16(tot16)
        pcum = plsc.cumsum(ptot16)
        cur_v[sl] = running + (pcum - ptot16)
        return running + pcum[15]

    lax.fori_loop(0, NBKT // 16, initgrp, 0)

    def rank(i, _):
        pk = pk_v[pl.ds(i * 16, 16)]
        b = lax.shift_right_logical(pk, 16 + BKT_SHIFT)
        rank16, last = plsc.scan_count(b)
        bse = plsc.load_gather(cur_v, [b])
        plsc.store_scatter(cur_v, [b], bse + rank16 + (1 - RANK_BASE),
                           mask=last)
        plsc.store_scatter(stg_v, [bse + rank16 - RANK_BASE], pk)
        return 0

    lax.fori_loop(0, EPT // 16, rank, 0)

    pltpu.sync_copy(
        stg_v, bucketed_hbm.at[pl.ds(pl.multiple_of(w * TSTRIDE, 16),
                                     TSTRIDE)])
    pltpu.sync_copy(cnt_v, counts_hbm.at[w])


def _seg_off(cnts_v, wp, b):
    lane = jnp.bitwise_and(b, 15)
    grp = lax.shift_right_logical(b, 4)
    lanes = lax.iota(jnp.int32, 16)

    def gbody(g, carry):
        running, start, plen, rcnt = carry
        row16 = cnts_v[wp, pl.ds(g * 16, 16)]
        p16 = _align16(row16)
        pcum = plsc.cumsum(p16)
        here = jnp.logical_and(g == grp, lanes == lane)
        start = start + jnp.sum(jnp.where(here, pcum - p16 + running, 0))
        plen = plen + jnp.sum(jnp.where(here, p16, 0))
        rcnt = rcnt + jnp.sum(jnp.where(here, row16, 0))
        return (running + pcum[15], start, plen, rcnt)

    _, start, plen, rcnt = lax.fori_loop(0, NBKT // 16, gbody, (0, 0, 0, 0))
    return start, plen, rcnt


@functools.partial(
    pl.kernel, mesh=_mesh,
    compiler_params=pltpu.CompilerParams(needs_layout_passes=False, use_tc_tiling_on_sc=False),
    out_type=jax.ShapeDtypeStruct((N_NODES, HID), jnp.bfloat16),
    scratch_types=[
        pltpu.VMEM((BKT_SZ + 1, HID), jnp.bfloat16),
        pltpu.VMEM((BKT_SZ + 1, HID), jnp.bfloat16),
        pltpu.VMEM((ECH + 32,), jnp.int32),
        pltpu.VMEM((ECH // GSUB, GSUB), jnp.int32),
        pltpu.VMEM((4, GSUB, HID), jnp.bfloat16),
        pltpu.VMEM((NW, NBKT), jnp.int32),
        pltpu.SemaphoreType.DMA,
    ],
)
def _e4_segmax(h_hbm, bucketed_hbm, counts_hbm, agg_hbm,
               agg_a, agg_b, ebuf_v, gidx_v, rows_v, cnts_v, sem):
    w = _wid()
    pltpu.sync_copy(counts_hbm, cnts_v)

    def bucket_body(rb, _):
        b = w + NW * rb

        def zrow(i, _):
            for q in range(HID // 32):
                agg_a[i, pl.ds(q * 32, 32)] = jnp.zeros((32,), jnp.bfloat16)
                agg_b[i, pl.ds(q * 32, 32)] = jnp.zeros((32,), jnp.bfloat16)
            return 0

        lax.fori_loop(0, BKT_SZ + 1, zrow, 0)

        def seg_body(wp, _):
            soff, plen, rcnt = _seg_off(cnts_v, wp, b)
            seg_base = wp * TSTRIDE + soff
            nch = lax.div(plen + (ECH - 1), ECH)

            def chunk_body(c, _):
                cbase = pl.multiple_of(seg_base + c * ECH, 8)
                pltpu.sync_copy(bucketed_hbm.at[pl.ds(cbase, ECH + 16)],
                                ebuf_v.at[pl.ds(0, ECH + 16)])
                rem = jnp.minimum(ECH, plen - c * ECH)
                erem = rcnt - c * ECH
                nsub = lax.div(rem + (GSUB - 1), GSUB)

                def build_idx(j):
                    def bi(i, _):
                        p16 = ebuf_v[pl.ds(j * GSUB + i * 16, 16)]
                        gidx_v[j, pl.ds(i * 16, 16)] = jnp.bitwise_and(
                            p16, 0xFFFF)
                        return 0
                    lax.fori_loop(0, GSUB // 16, bi, 0)

                def consume(j):
                    ngrp = lax.div(jnp.minimum(GSUB, rem - j * GSUB), 16)
                    jb = jnp.bitwise_and(j, 3)

                    lanes16 = lax.iota(jnp.int32, 16)

                    def cg(i, _):
                        p16 = ebuf_v[pl.ds(j * GSUB + i * 16, 16)]
                        d16 = jnp.bitwise_and(
                            lax.shift_right_logical(p16, 16), BKT_SZ - 1)
                        valid16 = (j * GSUB + i * 16 + lanes16) < erem
                        d16 = jnp.where(valid16, d16, BKT_SZ)
                        for l in range(16):
                            d = d16[l]
                            acc = agg_a if l % 2 == 0 else agg_b
                            for q in range(HID // 32):
                                sl = pl.ds(q * 32, 32)
                                acc[d, sl] = jnp.maximum(
                                    acc[d, sl], rows_v[jb, i * 16 + l, sl])
                        return 0

                    lax.fori_loop(0, ngrp, cg, 0)

                build_idx(0)
                pltpu.async_copy(h_hbm.at[gidx_v.at[0]], rows_v.at[0], sem)
                for pj in (1, 2):
                    @pl.when(pj < nsub)
                    def _prefill(pj=pj):
                        build_idx(pj)
                        pltpu.async_copy(
                            h_hbm.at[gidx_v.at[pj]], rows_v.at[pj], sem)

                def sub_body(j, _):
                    @pl.when(j + 3 < nsub)
                    def _issue_next():
                        jn = j + 3
                        build_idx(jn)
                        pltpu.async_copy(
                            h_hbm.at[gidx_v.at[jn]],
                            rows_v.at[jnp.bitwise_and(jn, 3)], sem)

                    pltpu.make_async_copy(
                        h_hbm.at[pl.ds(0, GSUB)],
                        rows_v.at[jnp.bitwise_and(j, 3)], sem).wait()
                    consume(j)
                    return 0

                lax.fori_loop(0, nsub, sub_body, 0)
                return 0

            lax.fori_loop(0, nch, chunk_body, 0)
            return 0

        lax.fori_loop(0, NW, seg_body, 0)

        def mrow(i, _):
            for q in range(HID // 32):
                sl = pl.ds(q * 32, 32)
                agg_a[i, sl] = jnp.maximum(agg_a[i, sl], agg_b[i, sl])
            return 0

        lax.fori_loop(0, BKT_SZ, mrow, 0)
        pltpu.sync_copy(agg_a.at[pl.ds(0, BKT_SZ)],
                        agg_hbm.at[pl.ds(b * BKT_SZ, BKT_SZ)])
        return 0

    lax.fori_loop(0, NBKT // NW, bucket_body, 0)


POLY_PER = N_POLY // NW
NEG_INIT = -1e30


@functools.partial(
    pl.kernel, mesh=_mesh,
    compiler_params=pltpu.CompilerParams(needs_layout_passes=False, use_tc_tiling_on_sc=False),
    out_type=jax.ShapeDtypeStruct((N_POLY, HID), jnp.float32),
    scratch_types=[
        pltpu.VMEM((N_NODES + 16,), jnp.int32),
        pltpu.VMEM((POLY_PER + 1, HID), jnp.float32),
        pltpu.VMEM((GSUB, HID), jnp.float32),
    ],
)
def _e5_poly(hf_hbm, cluster_hbm, poly_hbm, cl_v, agg_v, rows_v):
    w = _wid()
    pltpu.sync_copy(cluster_hbm, cl_v.at[pl.ds(0, N_NODES)])
    p0 = w * POLY_PER
    p1 = p0 + POLY_PER

    def lower_bound(target):
        def cond(st):
            return st[0] < st[1]

        lanes = lax.iota(jnp.int32, 16)

        def body(st):
            l, r = st
            m = lax.div(l + r, 2)
            m_al = pl.multiple_of(jnp.bitwise_and(m, -16), 16)
            v16 = cl_v[pl.ds(m_al, 16)]
            cm = jnp.sum(jnp.where(lanes == (m - m_al), v16, 0))
            go_right = cm < target
            return (jnp.where(go_right, m + 1, l), jnp.where(go_right, r, m))

        return lax.while_loop(cond, body, (0, N_NODES))[0]

    lo = lower_bound(p0)
    hi = lower_bound(p1)

    def zrow(i, _):
        for q in range(HID // 16):
            agg_v[i, pl.ds(q * 16, 16)] = jnp.full((16,), NEG_INIT, jnp.float32)
        return 0

    lax.fori_loop(0, POLY_PER + 1, zrow, 0)

    lo_al = pl.multiple_of(jnp.bitwise_and(lo, -16), 16)
    nch = lax.div(hi - lo_al + (GSUB - 1), GSUB)
    lanes = lax.iota(jnp.int32, 16)

    def chunk_body(c, _):
        cbase = pl.multiple_of(lo_al + c * GSUB, 16)
        pltpu.sync_copy(hf_hbm.at[pl.ds(cbase, GSUB)], rows_v)

        def cg(i, _):
            c16 = cl_v[pl.ds(cbase + i * 16, 16)]
            for l in range(16):
                n = cbase + i * 16 + l
                valid = jnp.logical_and(n >= lo, n < hi)
                d = jnp.where(valid, c16[l] - p0, POLY_PER)
                for q in range(HID // 16):
                    sl = pl.ds(q * 16, 16)
                    agg_v[d, sl] = jnp.maximum(agg_v[d, sl],
                                               rows_v[i * 16 + l, sl])
            return 0

        lax.fori_loop(0, GSUB // 16, cg, 0)
        return 0

    lax.fori_loop(0, nch, chunk_body, 0)
    pltpu.sync_copy(agg_v.at[pl.ds(0, POLY_PER)],
                    poly_hbm.at[pl.ds(p0, POLY_PER)])


def _ln_relu(h, g, beta):
    mu = jnp.mean(h, axis=-1, keepdims=True)
    var = jnp.var(h, axis=-1, keepdims=True)
    h = (h - mu) / jnp.sqrt(var + 1e-5) * g + beta
    return jax.nn.relu(h)


def _mlp0_body(x_ref, w_ref, b_ref, g_ref, bt_ref, o_ref, ob_ref):
    h = jnp.dot(x_ref[...], w_ref[...], preferred_element_type=jnp.float32)
    o = _ln_relu(h + b_ref[...], g_ref[...], bt_ref[...])
    o_ref[...] = o
    ob_ref[...] = o.astype(jnp.bfloat16)


def _mlp0(x, W0, b0, g0, beta0):
    grid = (N_NODES // ROWS_PER_TC_BLK,)
    return pl.pallas_call(
        _mlp0_body,
        grid=grid,
        in_specs=[
            pl.BlockSpec((ROWS_PER_TC_BLK, 8), lambda i: (i, 0)),
            pl.BlockSpec((8, HID), lambda i: (0, 0)),
            pl.BlockSpec((1, HID), lambda i: (0, 0)),
            pl.BlockSpec((1, HID), lambda i: (0, 0)),
            pl.BlockSpec((1, HID), lambda i: (0, 0)),
        ],
        out_specs=[pl.BlockSpec((ROWS_PER_TC_BLK, HID), lambda i: (i, 0)),
                   pl.BlockSpec((ROWS_PER_TC_BLK, HID), lambda i: (i, 0))],
        out_shape=[jax.ShapeDtypeStruct((N_NODES, HID), jnp.float32),
                   jax.ShapeDtypeStruct((N_NODES, HID), jnp.bfloat16)],
    )(x, W0, b0.reshape(1, HID), g0.reshape(1, HID), beta0.reshape(1, HID))


def _mlp12_body(h_ref, a_ref, wa_ref, wb_ref, b_ref, g_ref, bt_ref, o_ref,
                ob_ref):
    a = a_ref[...].astype(jnp.float32)
    h = (jnp.dot(h_ref[...], wa_ref[...], preferred_element_type=jnp.float32)
         + jnp.dot(a, wb_ref[...], preferred_element_type=jnp.float32))
    o = _ln_relu(h + b_ref[...], g_ref[...], bt_ref[...])
    o_ref[...] = o
    ob_ref[...] = o.astype(jnp.bfloat16)


def _mlp12(h, agg, W, b, g, beta):
    grid = (N_NODES // ROWS_PER_TC_BLK,)
    return pl.pallas_call(
        _mlp12_body,
        grid=grid,
        in_specs=[
            pl.BlockSpec((ROWS_PER_TC_BLK, HID), lambda i: (i, 0)),
            pl.BlockSpec((ROWS_PER_TC_BLK, HID), lambda i: (i, 0)),
            pl.BlockSpec((HID, HID), lambda i: (0, 0)),
            pl.BlockSpec((HID, HID), lambda i: (0, 0)),
            pl.BlockSpec((1, HID), lambda i: (0, 0)),
            pl.BlockSpec((1, HID), lambda i: (0, 0)),
            pl.BlockSpec((1, HID), lambda i: (0, 0)),
        ],
        out_specs=[pl.BlockSpec((ROWS_PER_TC_BLK, HID), lambda i: (i, 0)),
                   pl.BlockSpec((ROWS_PER_TC_BLK, HID), lambda i: (i, 0))],
        out_shape=[jax.ShapeDtypeStruct((N_NODES, HID), jnp.float32),
                   jax.ShapeDtypeStruct((N_NODES, HID), jnp.bfloat16)],
    )(h, agg, W[:HID], W[HID:], b.reshape(1, HID), g.reshape(1, HID),
      beta.reshape(1, HID))


HF_ROWS = N_NODES + 2048


def _final_body(h_ref, a_ref, wa_ref, wb_ref, b_ref, o_ref):
    a = a_ref[...].astype(jnp.float32)
    o_ref[...] = (
        jnp.dot(h_ref[...], wa_ref[...], preferred_element_type=jnp.float32)
        + jnp.dot(a, wb_ref[...], preferred_element_type=jnp.float32)
        + b_ref[...])


def _final(h, agg, Wf, bf):
    grid = (N_NODES // ROWS_PER_TC_BLK,)
    return pl.pallas_call(
        _final_body,
        grid=grid,
        in_specs=[
            pl.BlockSpec((ROWS_PER_TC_BLK, HID), lambda i: (i, 0)),
            pl.BlockSpec((ROWS_PER_TC_BLK, HID), lambda i: (i, 0)),
            pl.BlockSpec((HID, HID), lambda i: (0, 0)),
            pl.BlockSpec((HID, HID), lambda i: (0, 0)),
            pl.BlockSpec((1, HID), lambda i: (0, 0)),
        ],
        out_specs=pl.BlockSpec((ROWS_PER_TC_BLK, HID), lambda i: (i, 0)),
        out_shape=jax.ShapeDtypeStruct((HF_ROWS, HID), jnp.float32),
    )(h, agg, Wf[:HID], Wf[HID:], bf.reshape(1, HID))


def _attn_body(poly_ref, id_ref, wq1_ref, wq2_ref, bq_ref, wk1_ref, wk2_ref,
               bk_ref, wv1_ref, wv2_ref, bv_ref, vl_ref, out_ref):
    p = poly_ref[...]
    p = jnp.where(p < -1e29, 0.0, p)
    nrm = jnp.sqrt(jnp.sum(p * p, axis=-1, keepdims=True))
    p = p / jnp.maximum(nrm, 1e-12)
    idb = id_ref[...]

    def proj(w1, w2, bb):
        return (jnp.dot(p, w1[...], preferred_element_type=jnp.float32)
                + idb[:, 0:1] * w2[0:1, :] + idb[:, 1:2] * w2[1:2, :]
                + bb[...])

    q = proj(wq1_ref, wq2_ref, bq_ref)
    k = proj(wk1_ref, wk2_ref, bk_ref)
    v = proj(wv1_ref, wv2_ref, bv_ref)
    s = jnp.dot(q, k.T, preferred_element_type=jnp.float32) / jnp.sqrt(
        jnp.float32(HID))
    key_pos = jax.lax.broadcasted_iota(jnp.int32, (TSL, TSL), 1).astype(
        jnp.float32)
    mask = key_pos < vl_ref[0, 0, 0]
    s = jnp.where(mask, s, -1e9)
    s = s - jnp.max(s, axis=-1, keepdims=True)
    e = jnp.exp(s)
    attn = e / jnp.sum(e, axis=-1, keepdims=True)
    out_ref[0] = jnp.dot(attn, v, preferred_element_type=jnp.float32)


def _attention(poly, identifier, Wq, bq, Wk, bk, Wv, bv, vl_f):
    full = lambda shape: pl.BlockSpec(shape, lambda b: tuple(0 for _ in shape))
    return pl.pallas_call(
        _attn_body,
        grid=(BATCH,),
        in_specs=[
            pl.BlockSpec((TSL, HID), lambda b: (b, 0)),
            pl.BlockSpec((TSL, 2), lambda b: (b, 0)),
            full((HID, HID)), full((2, HID)), full((1, HID)),
            full((HID, HID)), full((2, HID)), full((1, HID)),
            full((HID, HID)), full((2, HID)), full((1, HID)),
            pl.BlockSpec((1, 1, 1), lambda b: (b, 0, 0)),
        ],
        out_specs=pl.BlockSpec((1, TSL, HID), lambda b: (b, 0, 0)),
        out_shape=jax.ShapeDtypeStruct((BATCH, TSL, HID), jnp.float32),
    )(poly, identifier, Wq[:HID], Wq[HID:], bq.reshape(1, HID),
      Wk[:HID], Wk[HID:], bk.reshape(1, HID),
      Wv[:HID], Wv[HID:], bv.reshape(1, HID), vl_f)


def kernel(x, edge_index, cluster, identifier, valid_len, time_step_len,
           W0, b0, g0, beta0, W1, b1, g1, beta1, W2, b2, g2, beta2,
           Wf, bf, Wq, bq, Wk, bk, Wv, bv):
    bucketed, counts = _e13_bucket(edge_index)

    h, hb = _mlp0(x, W0, b0, g0, beta0)
    agg = _e4_segmax(hb, bucketed, counts)
    h, hb = _mlp12(h, agg, W1, b1, g1, beta1)
    agg = _e4_segmax(hb, bucketed, counts)
    h, hb = _mlp12(h, agg, W2, b2, g2, beta2)
    agg = _e4_segmax(hb, bucketed, counts)
    hf = _final(h, agg, Wf, bf)

    poly = _e5_poly(hf, cluster)

    vl_f = jnp.minimum(valid_len, time_step_len).astype(jnp.float32).reshape(
        BATCH, 1, 1)
    return _attention(poly, identifier, Wq, bq, Wk, bk, Wv, bv, vl_f)
